# baseline (device time: 12923 ns/iter reference)
import jax
import jax.numpy as jnp
from jax import lax
from jax.experimental import pallas as pl
from jax.experimental.pallas import tpu as pltpu

T = 256
D = 512
V_LOCAL = 4096


def kernel(x, W, labels):
    def body(x_ref, w_ref, lbl_ref, out_ref, send_buf, recv_buf, send_sem, recv_sem):
        my_x = lax.axis_index("x")
        my_y = lax.axis_index("y")
        peer = (1 - my_x, my_y)

        barrier_sem = pltpu.get_barrier_semaphore()
        pl.semaphore_signal(
            barrier_sem, inc=1, device_id=peer, device_id_type=pl.DeviceIdType.MESH
        )

        xb = x_ref[...].astype(jnp.bfloat16)
        wb = w_ref[...].astype(jnp.bfloat16)
        logits = jnp.dot(xb, wb, preferred_element_type=jnp.float32)

        s = jnp.sum(
            jnp.exp(logits), axis=1, keepdims=True, dtype=jnp.float32
        )
        loc = lbl_ref[...][:, None] - my_x * V_LOCAL
        col = lax.broadcasted_iota(jnp.int32, (T, V_LOCAL), 1)
        ll = jnp.sum(
            jnp.where(col == loc, logits, 0.0),
            axis=1,
            keepdims=True,
            dtype=jnp.float32,
        )

        send_buf[:, 0:1] = s
        send_buf[:, 1:2] = ll

        pl.semaphore_wait(barrier_sem, 1)
        rdma = pltpu.make_async_remote_copy(
            src_ref=send_buf,
            dst_ref=recv_buf,
            send_sem=send_sem,
            recv_sem=recv_sem,
            device_id=peer,
            device_id_type=pl.DeviceIdType.MESH,
        )
        rdma.start()
        rdma.wait_recv()

        s_r = recv_buf[:, 0:1]
        ll_r = recv_buf[:, 1:2]
        nll = jnp.log(s + s_r) - (ll + ll_r)
        out_ref[...] = nll[:, 0]

        rdma.wait_send()

    out = pl.pallas_call(
        body,
        out_shape=jax.ShapeDtypeStruct((T,), jnp.float32),
        in_specs=[pl.BlockSpec(memory_space=pltpu.VMEM)] * 3,
        out_specs=pl.BlockSpec(memory_space=pltpu.VMEM),
        scratch_shapes=[
            pltpu.VMEM((T, 2), jnp.float32),
            pltpu.VMEM((T, 2), jnp.float32),
            pltpu.SemaphoreType.DMA,
            pltpu.SemaphoreType.DMA,
        ],
        compiler_params=pltpu.CompilerParams(collective_id=0),
    )(x, W, labels)
    return out


# device time: 12811 ns/iter; 1.0087x vs baseline; 1.0087x over previous
import jax
import jax.numpy as jnp
from jax import lax
from jax.experimental import pallas as pl
from jax.experimental.pallas import tpu as pltpu

T = 256
D = 512
V_LOCAL = 4096


def kernel(x, W, labels):
    def body(x_ref, w_ref, lbl_ref, out_ref, send_buf, recv_buf, send_sem, recv_sem):
        my_x = lax.axis_index("x")
        my_y = lax.axis_index("y")
        peer = (1 - my_x, my_y)

        barrier_sem = pltpu.get_barrier_semaphore()
        pl.semaphore_signal(
            barrier_sem, inc=1, device_id=peer, device_id_type=pl.DeviceIdType.MESH
        )

        xb = x_ref[...].astype(jnp.bfloat16)
        loc = lbl_ref[...][:, None] - my_x * V_LOCAL

        NCK = 8
        CK = V_LOCAL // NCK
        s = jnp.zeros((T, 1), jnp.float32)
        ll = jnp.zeros((T, 1), jnp.float32)
        for c in range(NCK):
            wb = w_ref[:, pl.ds(c * CK, CK)].astype(jnp.bfloat16)
            lg = jnp.dot(xb, wb, preferred_element_type=jnp.float32)
            s = s + jnp.sum(
                jnp.exp(lg), axis=1, keepdims=True, dtype=jnp.float32
            )
            col = lax.broadcasted_iota(jnp.int32, (T, CK), 1) + c * CK
            ll = ll + jnp.sum(
                jnp.where(col == loc, lg, 0.0),
                axis=1,
                keepdims=True,
                dtype=jnp.float32,
            )

        send_buf[:, 0:1] = s
        send_buf[:, 1:2] = ll

        pl.semaphore_wait(barrier_sem, 1)
        rdma = pltpu.make_async_remote_copy(
            src_ref=send_buf,
            dst_ref=recv_buf,
            send_sem=send_sem,
            recv_sem=recv_sem,
            device_id=peer,
            device_id_type=pl.DeviceIdType.MESH,
        )
        rdma.start()
        rdma.wait_recv()

        s_r = recv_buf[:, 0:1]
        ll_r = recv_buf[:, 1:2]
        nll = jnp.log(s + s_r) - (ll + ll_r)
        out_ref[...] = nll[:, 0]

        rdma.wait_send()

    out = pl.pallas_call(
        body,
        out_shape=jax.ShapeDtypeStruct((T,), jnp.float32),
        in_specs=[pl.BlockSpec(memory_space=pltpu.VMEM)] * 3,
        out_specs=pl.BlockSpec(memory_space=pltpu.VMEM),
        scratch_shapes=[
            pltpu.VMEM((T, 2), jnp.float32),
            pltpu.VMEM((T, 2), jnp.float32),
            pltpu.SemaphoreType.DMA,
            pltpu.SemaphoreType.DMA,
        ],
        compiler_params=pltpu.CompilerParams(collective_id=0),
    )(x, W, labels)
    return out
